# baseline (speedup 1.0000x reference)
import functools

import numpy as np
import jax
import jax.numpy as jnp
from jax import lax
from jax.experimental import pallas as pl
from jax.experimental.pallas import tpu as pltpu
from jax.experimental.pallas import tpu_sc as plsc

N, C = 4096, 256
C2 = 2 * C
K = 16
NSC, NSUB = 2, 16
NW = NSC * NSUB
RPW = N // NW
RB = 256
NB = N // RB
NG = 256
GM = N // NG


def _stage_ab_body(x_ref, w1_ref, b1_ref, wg_ref, bg_ref,
                   u_ref, v_ref, dist_ref, cm_ref, am_ref,
                   zn_s, sqr_s):
    i = pl.program_id(0)

    @pl.when(i == 0)
    def _():
        x = x_ref[...]
        h = jnp.maximum(
            lax.dot_general(x, w1_ref[...], (((1,), (1,)), ((), ())),
                            preferred_element_type=jnp.float32)
            + b1_ref[...], 0.0)
        cs = jnp.sum(h * h, axis=0, keepdims=True)
        zn = h / jnp.sqrt(cs)
        zn_s[...] = zn
        sqr_s[...] = lax.dot_general(jnp.ones((1, C), jnp.float32), zn * zn,
                                     (((1,), (1,)), ((), ())),
                                     precision=lax.Precision.HIGHEST,
                                     preferred_element_type=jnp.float32)
        wg = wg_ref[...]
        u_ref[...] = lax.dot_general(h, wg[:, :C] - wg[:, C:],
                                     (((1,), (1,)), ((), ())),
                                     preferred_element_type=jnp.float32
                                     ) + bg_ref[...]
        v = lax.dot_general(h, wg[:, C:], (((1,), (1,)), ((), ())),
                            preferred_element_type=jnp.float32)
        va = lax.bitcast_convert_type(v[:, :C].astype(jnp.bfloat16),
                                      jnp.uint16)
        vb = lax.bitcast_convert_type(v[:, C:].astype(jnp.bfloat16),
                                      jnp.uint16)
        w = va.astype(jnp.uint32) | (vb.astype(jnp.uint32) << 16)
        v_ref[...] = lax.bitcast_convert_type(w, jnp.float32)

    @pl.when(i > 0)
    def _():
        blk = i - 1
        zn_blk = zn_s[pl.ds(blk * RB, RB), :]
        dg = lax.dot_general(zn_blk, zn_s[...], (((1,), (1,)), ((), ())),
                             preferred_element_type=jnp.float32)
        sq_blk = jnp.sum(zn_blk * zn_blk, axis=1, keepdims=True)
        dist = sq_blk - 2.0 * dg + sqr_s[...]
        dist_ref[...] = dist
        m = dist[:, :NG]
        am = jnp.zeros((RB, NG), jnp.int32)
        for t in range(1, GM):
            sl = dist[:, t * NG:(t + 1) * NG]
            lt = sl < m
            m = jnp.where(lt, sl, m)
            am = jnp.where(lt, t, am)
        cm_ref[...] = m
        am_ref[...] = am


def _stage_ab(x, w1, b1, wg, bg):
    zero = lambda i: (0, 0)
    prev = lambda i: (jnp.maximum(i - 1, 0), 0)
    return pl.pallas_call(
        _stage_ab_body,
        grid=(NB + 1,),
        in_specs=[
            pl.BlockSpec((N, C), zero),
            pl.BlockSpec((C, C), zero),
            pl.BlockSpec((1, C), zero),
            pl.BlockSpec((C2, C2), zero),
            pl.BlockSpec((1, C2), zero),
        ],
        out_specs=(
            pl.BlockSpec((N, C2), zero),
            pl.BlockSpec((N, C), zero),
            pl.BlockSpec((RB, N), prev),
            pl.BlockSpec((RB, NG), prev),
            pl.BlockSpec((RB, NG), prev),
        ),
        out_shape=(
            jax.ShapeDtypeStruct((N, C2), jnp.float32),
            jax.ShapeDtypeStruct((N, C), jnp.float32),
            jax.ShapeDtypeStruct((N, N), jnp.float32),
            jax.ShapeDtypeStruct((N, NG), jnp.float32),
            jax.ShapeDtypeStruct((N, NG), jnp.int32),
        ),
        scratch_shapes=[
            pltpu.VMEM((N, C), jnp.float32),
            pltpu.VMEM((1, N), jnp.float32),
        ],
    )(x, w1, b1, wg, bg)


def _merge_sorted(t, ti, v, vi):
    rv = jnp.flip(v)
    ri = jnp.flip(vi)
    m = t <= rv
    nt = jnp.where(m, t, rv)
    ni = jnp.where(m, ti, ri)
    return plsc.sort_key_val(nt, ni)


def _sc_body(dist_hbm, cm_hbm, am_hbm, v_hbm, m_hbm,
             cm_v, am_v, dist_v, vrows, out_v,
             semd0, semd1, semg0, semg1, semg2, semg3, semsl):
    wid = lax.axis_index("s") * NSC + lax.axis_index("c")
    base = wid * RPW
    pltpu.async_copy(cm_hbm.at[pl.ds(base, RPW)], cm_v, semsl)
    pltpu.async_copy(am_hbm.at[pl.ds(base, RPW)], am_v, semsl)
    pltpu.async_copy(dist_hbm.at[base], dist_v.at[0], semd0)
    pltpu.async_copy(dist_hbm.at[base + 1], dist_v.at[1], semd1)
    pltpu.make_async_copy(cm_hbm.at[pl.ds(base, RPW)], cm_v, semsl).wait()
    pltpu.make_async_copy(am_hbm.at[pl.ds(base, RPW)], am_v, semsl).wait()
    i16 = lax.iota(jnp.int32, 16)
    inf = jnp.float32(jnp.inf)
    sems = (semd0, semd1)
    semg = (semg0, semg1)

    def topk(r, b):
        lvl = []
        for s in range(NG // 16):
            v = cm_v[r, pl.ds(s * 16, 16)]
            lvl.append(plsc.sort_key_val(v, s * 16 + i16))
        while len(lvl) > 1:
            lvl = [_merge_sorted(a[0], a[1], c[0], c[1])
                   for a, c in zip(lvl[::2], lvl[1::2])]
        gt, sg = lvl[0]

        rvec = jnp.full((16,), r, jnp.int32)
        amv = plsc.load_gather(am_v, [rvec, sg])
        ti0 = sg + amv * NG
        bvec = jnp.full((16,), b, jnp.int32)
        lvl = []
        for s in range(GM):
            gidx = sg + s * NG
            v = plsc.load_gather(dist_v, [bvec, gidx])
            v = jnp.where(amv == s, inf, v)
            lvl.append(plsc.sort_key_val(v, gidx))
        while len(lvl) > 1:
            lvl = [_merge_sorted(a[0], a[1], c[0], c[1])
                   for a, c in zip(lvl[::2], lvl[1::2])]
        _, ti = _merge_sorted(gt, ti0, lvl[0][0], lvl[0][1])
        return ti

    def reduce_prev(rp, gb):
        for cc in range(C // 16):
            off = cc * 16
            m = plsc.bitcast(vrows[gb, 0, pl.ds(off, 16)], jnp.bfloat16)
            for k in range(1, K):
                m = jnp.maximum(
                    m, plsc.bitcast(vrows[gb, k, pl.ds(off, 16)],
                                    jnp.bfloat16))
            out_v[rp, pl.ds(off, 16)] = plsc.bitcast(m, jnp.float32)

    def drain_gather(gb):
        pltpu.make_async_copy(v_hbm.at[pl.ds(0, K)], vrows.at[gb],
                              semg[gb]).wait()

    def step(r, b, prefetch, drain):
        pltpu.make_async_copy(dist_hbm.at[base + r], dist_v.at[b],
                              sems[b]).wait()
        ti = topk(r, b)
        if prefetch:
            pltpu.async_copy(dist_hbm.at[base + r + 2], dist_v.at[b], sems[b])
        pltpu.async_copy(v_hbm.at[ti], vrows.at[b], semg[b])
        if drain:
            drain_gather(1 - b)
            reduce_prev(r - 1, 1 - b)

    step(0, 0, True, False)
    step(1, 1, True, True)

    def rbody(i, _):
        r0 = i * 2
        step(r0, 0, True, True)
        step(r0 + 1, 1, True, True)
        return 0

    lax.fori_loop(1, RPW // 2 - 1, rbody, 0)
    step(RPW - 2, 0, False, True)
    step(RPW - 1, 1, False, True)
    drain_gather(1)
    reduce_prev(RPW - 1, 1)
    pltpu.sync_copy(out_v, m_hbm.at[pl.ds(base, RPW)])


def _sc_stage(dist, cm, am, v):
    mesh = plsc.VectorSubcoreMesh(core_axis_name="c", subcore_axis_name="s")
    f = pl.kernel(
        _sc_body,
        out_type=jax.ShapeDtypeStruct((N, C), jnp.float32),
        mesh=mesh,
        compiler_params=pltpu.CompilerParams(needs_layout_passes=False),
        scratch_types=[
            pltpu.VMEM((RPW, NG), jnp.float32),
            pltpu.VMEM((RPW, NG), jnp.int32),
            pltpu.VMEM((2, N), jnp.float32),
            pltpu.VMEM((2, K, C), jnp.float32),
            pltpu.VMEM((RPW, C), jnp.float32),
            pltpu.SemaphoreType.DMA,
            pltpu.SemaphoreType.DMA,
            pltpu.SemaphoreType.DMA,
            pltpu.SemaphoreType.DMA,
            pltpu.SemaphoreType.DMA,
            pltpu.SemaphoreType.DMA,
            pltpu.SemaphoreType.DMA,
        ],
    )
    return f(dist, cm, am, v)


def _stage_e_body(u_ref, m_ref, w2_ref, b2_ref, x_ref, out_ref):
    w = lax.bitcast_convert_type(m_ref[...], jnp.uint32)
    lo = lax.bitcast_convert_type((w & 0xFFFF).astype(jnp.uint16),
                                  jnp.bfloat16).astype(jnp.float32)
    hi = lax.bitcast_convert_type((w >> 16).astype(jnp.uint16),
                                  jnp.bfloat16).astype(jnp.float32)
    u = u_ref[...]
    w2 = w2_ref[...]
    g_lo = jnp.maximum(u[:, :C] + lo, 0.0)
    g_hi = jnp.maximum(u[:, C:] + hi, 0.0)
    out = lax.dot_general(g_lo, w2[:, :C], (((1,), (1,)), ((), ())),
                          preferred_element_type=jnp.float32)
    out += lax.dot_general(g_hi, w2[:, C:], (((1,), (1,)), ((), ())),
                           preferred_element_type=jnp.float32)
    out_ref[...] = jnp.maximum(out + b2_ref[...], 0.0) + x_ref[...]


def _stage_e(u, m, w2, b2, x):
    return pl.pallas_call(
        _stage_e_body,
        out_shape=jax.ShapeDtypeStruct((N, C), jnp.float32),
    )(u, m, w2, b2, x)


@jax.jit
def kernel(x, W1, b1, Wg, bg, W2, b2):
    u, vp, dist, cm, am = _stage_ab(x, W1, b1.reshape(1, C), Wg,
                                    bg.reshape(1, C2))
    m = _sc_stage(dist, cm, am, vp)
    return _stage_e(u, m, W2, b2.reshape(1, C), x)


# speedup vs baseline: 1.0127x; 1.0127x over previous
import functools

import numpy as np
import jax
import jax.numpy as jnp
from jax import lax
from jax.experimental import pallas as pl
from jax.experimental.pallas import tpu as pltpu
from jax.experimental.pallas import tpu_sc as plsc

N, C = 4096, 256
C2 = 2 * C
K = 16
NSC, NSUB = 2, 16
NW = NSC * NSUB
RPW = N // NW
RB = 256
NB = N // RB
NG = 256
GM = N // NG


def _stage_ab_body(x_ref, w1_ref, b1_ref, wg_ref, bg_ref,
                   u_ref, v_ref, dist_ref, cm_ref, am_ref,
                   zn_s, sqr_s):
    i = pl.program_id(0)

    @pl.when(i == 0)
    def _():
        x = x_ref[...]
        h = jnp.maximum(
            lax.dot_general(x, w1_ref[...], (((1,), (1,)), ((), ())),
                            preferred_element_type=jnp.float32)
            + b1_ref[...], 0.0)
        cs = jnp.sum(h * h, axis=0, keepdims=True)
        zn = h / jnp.sqrt(cs)
        zn_s[...] = zn
        sqr_s[...] = lax.dot_general(jnp.ones((1, C), jnp.float32), zn * zn,
                                     (((1,), (1,)), ((), ())),
                                     precision=lax.Precision.HIGHEST,
                                     preferred_element_type=jnp.float32)
        wg = wg_ref[...]
        u_ref[...] = lax.dot_general(h, wg[:, :C] - wg[:, C:],
                                     (((1,), (1,)), ((), ())),
                                     preferred_element_type=jnp.float32
                                     ) + bg_ref[...]
        v = lax.dot_general(h, wg[:, C:], (((1,), (1,)), ((), ())),
                            preferred_element_type=jnp.float32)
        va = lax.bitcast_convert_type(v[:, :C].astype(jnp.bfloat16),
                                      jnp.uint16)
        vb = lax.bitcast_convert_type(v[:, C:].astype(jnp.bfloat16),
                                      jnp.uint16)
        w = va.astype(jnp.uint32) | (vb.astype(jnp.uint32) << 16)
        v_ref[...] = lax.bitcast_convert_type(w, jnp.float32)

    @pl.when(i > 0)
    def _():
        blk = i - 1
        zn_blk = zn_s[pl.ds(blk * RB, RB), :]
        dg = lax.dot_general(zn_blk, zn_s[...], (((1,), (1,)), ((), ())),
                             preferred_element_type=jnp.float32)
        sq_blk = jnp.sum(zn_blk * zn_blk, axis=1, keepdims=True)
        dist = sq_blk - 2.0 * dg + sqr_s[...]
        dist_ref[...] = dist
        m = dist[:, :NG]
        am = jnp.zeros((RB, NG), jnp.int32)
        for t in range(1, GM):
            sl = dist[:, t * NG:(t + 1) * NG]
            lt = sl < m
            m = jnp.where(lt, sl, m)
            am = jnp.where(lt, t, am)
        cm_ref[...] = m
        am_ref[...] = am


def _stage_ab(x, w1, b1, wg, bg):
    zero = lambda i: (0, 0)
    prev = lambda i: (jnp.maximum(i - 1, 0), 0)
    return pl.pallas_call(
        _stage_ab_body,
        grid=(NB + 1,),
        in_specs=[
            pl.BlockSpec((N, C), zero),
            pl.BlockSpec((C, C), zero),
            pl.BlockSpec((1, C), zero),
            pl.BlockSpec((C2, C2), zero),
            pl.BlockSpec((1, C2), zero),
        ],
        out_specs=(
            pl.BlockSpec((N, C2), zero),
            pl.BlockSpec((N, C), zero),
            pl.BlockSpec((RB, N), prev),
            pl.BlockSpec((RB, NG), prev),
            pl.BlockSpec((RB, NG), prev),
        ),
        out_shape=(
            jax.ShapeDtypeStruct((N, C2), jnp.float32),
            jax.ShapeDtypeStruct((N, C), jnp.float32),
            jax.ShapeDtypeStruct((N, N), jnp.float32),
            jax.ShapeDtypeStruct((N, NG), jnp.float32),
            jax.ShapeDtypeStruct((N, NG), jnp.int32),
        ),
        scratch_shapes=[
            pltpu.VMEM((N, C), jnp.float32),
            pltpu.VMEM((1, N), jnp.float32),
        ],
    )(x, w1, b1, wg, bg)


def _merge_sorted(t, ti, v, vi):
    rv = jnp.flip(v)
    ri = jnp.flip(vi)
    m = t <= rv
    nt = jnp.where(m, t, rv)
    ni = jnp.where(m, ti, ri)
    return plsc.sort_key_val(nt, ni)


def _sc_body(dist_hbm, cm_hbm, am_hbm, v_hbm, m_hbm,
             cm_v, am_v, dist_v, vrows, out_v,
             semd0, semd1, semg0, semg1, semg2, semg3, semsl):
    wid = lax.axis_index("s") * NSC + lax.axis_index("c")
    base = wid * RPW
    pltpu.async_copy(cm_hbm.at[pl.ds(base, RPW)], cm_v, semsl)
    pltpu.async_copy(am_hbm.at[pl.ds(base, RPW)], am_v, semsl)
    pltpu.async_copy(dist_hbm.at[base], dist_v.at[0], semd0)
    pltpu.async_copy(dist_hbm.at[base + 1], dist_v.at[1], semd1)
    pltpu.make_async_copy(cm_hbm.at[pl.ds(base, RPW)], cm_v, semsl).wait()
    pltpu.make_async_copy(am_hbm.at[pl.ds(base, RPW)], am_v, semsl).wait()
    i16 = lax.iota(jnp.int32, 16)
    inf = jnp.float32(jnp.inf)
    sems = (semd0, semd1)
    semg = (semg0, semg1)

    def topk(r, b):
        lvl = []
        for s in range(NG // 16):
            v = cm_v[r, pl.ds(s * 16, 16)]
            lvl.append(plsc.sort_key_val(v, s * 16 + i16))
        while len(lvl) > 1:
            lvl = [_merge_sorted(a[0], a[1], c[0], c[1])
                   for a, c in zip(lvl[::2], lvl[1::2])]
        gt, sg = lvl[0]

        rvec = jnp.full((16,), r, jnp.int32)
        amv = plsc.load_gather(am_v, [rvec, sg])
        ti0 = sg + amv * NG
        bvec = jnp.full((16,), b, jnp.int32)
        lvl = []
        for s in range(GM):
            gidx = sg + s * NG
            v = plsc.load_gather(dist_v, [bvec, gidx])
            v = jnp.where(amv == s, inf, v)
            lvl.append(plsc.sort_key_val(v, gidx))
        while len(lvl) > 1:
            lvl = [_merge_sorted(a[0], a[1], c[0], c[1])
                   for a, c in zip(lvl[::2], lvl[1::2])]
        _, ti = _merge_sorted(gt, ti0, lvl[0][0], lvl[0][1])
        return ti

    def reduce_prev(rp, gb):
        for cc in range(C // 16):
            off = cc * 16
            m = plsc.bitcast(vrows[gb, 0, pl.ds(off, 16)], jnp.bfloat16)
            for k in range(1, K):
                m = jnp.maximum(
                    m, plsc.bitcast(vrows[gb, k, pl.ds(off, 16)],
                                    jnp.bfloat16))
            out_v[rp, pl.ds(off, 16)] = plsc.bitcast(m, jnp.float32)

    def drain_gather(gb):
        pltpu.make_async_copy(v_hbm.at[pl.ds(0, K)], vrows.at[gb],
                              semg[gb]).wait()

    def rbody(i, _):
        r0 = i * 2
        for u in range(2):
            r = r0 + u
            b = u
            nb = 1 - u
            pltpu.make_async_copy(dist_hbm.at[base + r], dist_v.at[b],
                                  sems[b]).wait()
            ti = topk(r, b)

            @pl.when(r + 2 < RPW)
            def _():
                pltpu.async_copy(dist_hbm.at[base + r + 2], dist_v.at[b],
                                 sems[b])

            pltpu.async_copy(v_hbm.at[ti], vrows.at[b], semg[b])

            @pl.when(r >= 1)
            def _():
                drain_gather(nb)
                reduce_prev(r - 1, nb)
        return 0

    lax.fori_loop(0, RPW // 2, rbody, 0)
    drain_gather((RPW - 1) % 2)
    reduce_prev(RPW - 1, (RPW - 1) % 2)
    pltpu.sync_copy(out_v, m_hbm.at[pl.ds(base, RPW)])


def _sc_stage(dist, cm, am, v):
    mesh = plsc.VectorSubcoreMesh(core_axis_name="c", subcore_axis_name="s")
    f = pl.kernel(
        _sc_body,
        out_type=jax.ShapeDtypeStruct((N, C), jnp.float32),
        mesh=mesh,
        compiler_params=pltpu.CompilerParams(needs_layout_passes=False),
        scratch_types=[
            pltpu.VMEM((RPW, NG), jnp.float32),
            pltpu.VMEM((RPW, NG), jnp.int32),
            pltpu.VMEM((2, N), jnp.float32),
            pltpu.VMEM((2, K, C), jnp.float32),
            pltpu.VMEM((RPW, C), jnp.float32),
            pltpu.SemaphoreType.DMA,
            pltpu.SemaphoreType.DMA,
            pltpu.SemaphoreType.DMA,
            pltpu.SemaphoreType.DMA,
            pltpu.SemaphoreType.DMA,
            pltpu.SemaphoreType.DMA,
            pltpu.SemaphoreType.DMA,
        ],
    )
    return f(dist, cm, am, v)


def _stage_e_body(u_ref, m_ref, w2_ref, b2_ref, x_ref, out_ref):
    w = lax.bitcast_convert_type(m_ref[...], jnp.uint32)
    lo = lax.bitcast_convert_type((w & 0xFFFF).astype(jnp.uint16),
                                  jnp.bfloat16).astype(jnp.float32)
    hi = lax.bitcast_convert_type((w >> 16).astype(jnp.uint16),
                                  jnp.bfloat16).astype(jnp.float32)
    u = u_ref[...]
    w2 = w2_ref[...]
    g_lo = jnp.maximum(u[:, :C] + lo, 0.0)
    g_hi = jnp.maximum(u[:, C:] + hi, 0.0)
    out = lax.dot_general(g_lo, w2[:, :C], (((1,), (1,)), ((), ())),
                          preferred_element_type=jnp.float32)
    out += lax.dot_general(g_hi, w2[:, C:], (((1,), (1,)), ((), ())),
                           preferred_element_type=jnp.float32)
    out_ref[...] = jnp.maximum(out + b2_ref[...], 0.0) + x_ref[...]


def _stage_e(u, m, w2, b2, x):
    return pl.pallas_call(
        _stage_e_body,
        out_shape=jax.ShapeDtypeStruct((N, C), jnp.float32),
    )(u, m, w2, b2, x)


@jax.jit
def kernel(x, W1, b1, Wg, bg, W2, b2):
    u, vp, dist, cm, am = _stage_ab(x, W1, b1.reshape(1, C), Wg,
                                    bg.reshape(1, C2))
    m = _sc_stage(dist, cm, am, vp)
    return _stage_e(u, m, W2, b2.reshape(1, C), x)
